# baseline (speedup 1.0000x reference)
import functools

import jax
import jax.numpy as jnp
from jax import lax
from jax.experimental import pallas as pl
from jax.experimental.pallas import tpu as pltpu
from jax.experimental.pallas import tpu_sc as plsc

EMBED = 256
H = 8
HD = 32
SP = 16
SHAPES = ((100, 100), (50, 50), (25, 25), (13, 13))
STARTS = (0, 10000, 12500, 13125)
LEN_V = 13294
QB = 512

NQ = 2
PAIRS_PER_W = 512


def _prep_body(q_ref, r_ref, woff_ref, boff_ref, wattn_ref, battn_ref,
               idx_ref, w_ref):
    b = pl.program_id(0)
    q = q_ref[0]

    offs = jnp.dot(q, woff_ref[...], preferred_element_type=jnp.float32)
    offs = offs + boff_ref[...]
    off_x = offs[:, :128]
    off_y = offs[:, 128:]

    logits = jnp.dot(q, wattn_ref[...], preferred_element_type=jnp.float32)
    logits = logits + battn_ref[...]
    e = jnp.exp(logits)
    ri = lax.broadcasted_iota(jnp.int32, (128, 128), 0) // SP
    ci = lax.broadcasted_iota(jnp.int32, (128, 128), 1) // SP
    bd = (ri == ci).astype(jnp.float32)
    denom = jnp.dot(e, bd, preferred_element_type=jnp.float32)
    attn = e / denom

    lane = lax.broadcasted_iota(jnp.int32, (QB, 128), 1)
    h_l = lane // SP
    lvl = (lane // 4) % 4
    w_i = jnp.where(lvl == 0, SHAPES[0][1],
                    jnp.where(lvl == 1, SHAPES[1][1],
                              jnp.where(lvl == 2, SHAPES[2][1], SHAPES[3][1])))
    h_i = w_i
    start_v = jnp.where(lvl == 0, STARTS[0],
                        jnp.where(lvl == 1, STARTS[1],
                                  jnp.where(lvl == 2, STARTS[2], STARTS[3])))
    w_f = w_i.astype(jnp.float32)
    h_f = h_i.astype(jnp.float32)

    r = r_ref[0]
    rx = r[:, 0:1]
    ry = r[:, 1:2]
    rw = r[:, 2:3]
    rh = r[:, 3:4]

    loc_x = rx + off_x * 0.125 * rw
    loc_y = ry + off_y * 0.125 * rh
    x = loc_x * w_f - 0.5
    y = loc_y * h_f - 0.5
    x0f = jnp.floor(x)
    y0f = jnp.floor(y)
    fx = x - x0f
    fy = y - y0f
    x0 = x0f.astype(jnp.int32)
    y0 = y0f.astype(jnp.int32)

    base_row = (b * LEN_V + start_v) * H + h_l
    idx_parts = []
    w_parts = []
    for dx, dy in ((0, 0), (1, 0), (0, 1), (1, 1)):
        xi = x0 + dx
        yi = y0 + dy
        valid = (xi >= 0) & (xi < w_i) & (yi >= 0) & (yi < h_i)
        xc = jnp.clip(xi, 0, w_i - 1)
        yc = jnp.clip(yi, 0, h_i - 1)
        flat = base_row + (yc * w_i + xc) * H
        wx = fx if dx else (1.0 - fx)
        wy = fy if dy else (1.0 - fy)
        wgt = attn * wx * wy * valid.astype(jnp.float32)
        idx_parts.append(flat)
        w_parts.append(wgt)
    idx_ref[0] = jnp.concatenate(idx_parts, axis=-1)
    w_ref[0] = jnp.concatenate(w_parts, axis=-1)


def _prep(query, rp, woff_p, boff_p, wattn, battn2):
    bs, lq, _ = query.shape
    grid = (bs, lq // QB)
    return pl.pallas_call(
        _prep_body,
        grid=grid,
        in_specs=[
            pl.BlockSpec((1, QB, EMBED), lambda b, i: (b, i, 0)),
            pl.BlockSpec((1, QB, 4), lambda b, i: (b, i, 0)),
            pl.BlockSpec((EMBED, EMBED), lambda b, i: (0, 0)),
            pl.BlockSpec((1, EMBED), lambda b, i: (0, 0)),
            pl.BlockSpec((EMBED, 128), lambda b, i: (0, 0)),
            pl.BlockSpec((1, 128), lambda b, i: (0, 0)),
        ],
        out_specs=[
            pl.BlockSpec((1, QB, 512), lambda b, i: (b, i, 0)),
            pl.BlockSpec((1, QB, 512), lambda b, i: (b, i, 0)),
        ],
        out_shape=[
            jax.ShapeDtypeStruct((bs, lq, 512), jnp.int32),
            jax.ShapeDtypeStruct((bs, lq, 512), jnp.float32),
        ],
    )(query, rp, woff_p, boff_p, wattn, battn2)


NSTEPS = PAIRS_PER_W // NQ


def _bag_body(value_ref, idx_hbm, w_hbm, out_hbm, idx_v, w_v, rows_v, out_v,
              ss0, ss1, ss2, ss3, gs0, gs1, os0, os1):
    ssem = (ss0, ss1, ss2, ss3)
    gsem = (gs0, gs1)
    osem = (os0, os1)
    nc = lax.axis_index("c")
    ns = lax.axis_index("s")
    wid = ns * 2 + nc
    pair0 = wid * PAIRS_PER_W

    def stage(s, sb, sem):
        g = pair0 + NQ * s
        return (
            pltpu.make_async_copy(idx_hbm.at[pl.ds(4 * g, 8)],
                                  idx_v.at[pl.ds(sb * 8, 8)], sem),
            pltpu.make_async_copy(w_hbm.at[pl.ds(g * 512, 1024)],
                                  w_v.at[pl.ds(sb * 1024, 1024)], sem),
        )

    def gathers(sb, rb, sem):
        return [pltpu.make_async_copy(value_ref.at[idx_v.at[sb * 8 + j]],
                                      rows_v.at[rb * 8 + j], sem)
                for j in range(8)]

    def out_dma(s, rb, sem):
        g = pair0 + NQ * s
        return pltpu.make_async_copy(out_v.at[pl.ds(rb * 512, 512)],
                                     out_hbm.at[pl.ds(g * 256, 512)], sem)

    lane = lax.iota(jnp.int32, 16)
    qv = lane // 8
    hv = lane % 8
    wlane = qv * 512 + hv * 16
    jlane = qv * 4
    rlane = hv * 16
    obase = lane * 32
    chvs = [jnp.full((16,), ch, jnp.int32) for ch in range(32)]

    def compute(s, sb, rb):
        def kstep(k, accs):
            c = k // 16
            lp = k % 16
            wv = plsc.load_gather(
                w_v, [wlane + (sb * 1024 + c * 128 + lp)])
            jv = jlane + (rb * 8 + c)
            rv = rlane + lp
            new = []
            for ch in range(32):
                g = plsc.load_gather(rows_v, [jv, rv, chvs[ch]])
                new.append(accs[ch] + wv * g)
            return tuple(new)

        accs = lax.fori_loop(0, 64, kstep,
                             tuple(jnp.zeros((16,), jnp.float32)
                                   for _ in range(32)))
        for ch in range(32):
            plsc.store_scatter(out_v, [obase + (rb * 512 + ch)], accs[ch])

    for d in stage(0, 0, ssem[0]):
        d.start()
        d.wait()
    for d in gathers(0, 0, gsem[0]):
        d.start()
    for d in stage(1, 1, ssem[1]):
        d.start()

    def body4(tt, _):
        for i in range(4):
            s = tt * 4 + i
            sb1 = (i + 1) % 4
            sb2 = (i + 2) % 4
            rb = i % 2
            rb1 = (i + 1) % 2

            @pl.when(s + 1 < NSTEPS)
            def _():
                for d in stage(s + 1, sb1, ssem[sb1]):
                    d.wait()
                for d in gathers(sb1, rb1, gsem[rb1]):
                    d.start()

            for d in gathers(i, rb, gsem[rb]):
                d.wait()

            @pl.when(s + 2 < NSTEPS)
            def _():
                for d in stage(s + 2, sb2, ssem[sb2]):
                    d.start()

            @pl.when(s >= 2)
            def _():
                out_dma(s, rb, osem[rb]).wait()

            compute(s, i, rb)
            out_dma(s, rb, osem[rb]).start()
        return 0

    lax.fori_loop(0, NSTEPS // 4, body4, 0)
    out_dma(0, 0, osem[0]).wait()
    out_dma(0, 1, osem[1]).wait()


def _bag_sum(value2, idx2, w2):
    mesh = plsc.VectorSubcoreMesh(core_axis_name="c", subcore_axis_name="s")
    f = pl.kernel(
        _bag_body,
        out_type=jax.ShapeDtypeStruct((16384 * H * HD,), jnp.float32),
        mesh=mesh,
        scratch_types=[
            pltpu.VMEM((32, 128), jnp.int32),
            pltpu.VMEM((4096,), jnp.float32),
            pltpu.VMEM((16, 128, HD), jnp.float32),
            pltpu.VMEM((1024,), jnp.float32),
            pltpu.SemaphoreType.DMA,
            pltpu.SemaphoreType.DMA,
            pltpu.SemaphoreType.DMA,
            pltpu.SemaphoreType.DMA,
            pltpu.SemaphoreType.DMA,
            pltpu.SemaphoreType.DMA,
            pltpu.SemaphoreType.DMA,
            pltpu.SemaphoreType.DMA,
        ],
        compiler_params=pltpu.CompilerParams(needs_layout_passes=False,
                                             use_tc_tiling_on_sc=False),
    )
    return f(value2, idx2, w2)


def kernel(query, reference_points, value, value_spatial_shapes,
           W_off, b_off, W_attn, b_attn):
    bs, lq = query.shape[:2]
    woff_p = jnp.concatenate([W_off[:, 0::2], W_off[:, 1::2]], axis=1)
    boff_p = jnp.concatenate([b_off[0::2], b_off[1::2]])[None, :]
    battn2 = b_attn[None, :]
    rp = reference_points.reshape(bs, lq, 4)

    idx, w = _prep(query, rp, woff_p, boff_p, W_attn, battn2)

    value2 = value.reshape(bs * LEN_V * H, HD)
    idx2 = idx.reshape(bs * 4 * lq, 128)
    w2 = w.reshape(bs * 4 * lq * 128)
    out2 = _bag_sum(value2, idx2, w2)
    return out2.reshape(bs, lq, EMBED)


# speedup vs baseline: 4.5156x; 4.5156x over previous
import functools

import jax
import jax.numpy as jnp
from jax import lax
from jax.experimental import pallas as pl
from jax.experimental.pallas import tpu as pltpu
from jax.experimental.pallas import tpu_sc as plsc

EMBED = 256
H = 8
HD = 32
SP = 16
SHAPES = ((100, 100), (50, 50), (25, 25), (13, 13))
STARTS = (0, 10000, 12500, 13125)
LEN_V = 13294
QB = 512

NQ = 2
PAIRS_PER_W = 512


def _prep_body(q_ref, r_ref, woff_ref, boff_ref, wattn_ref, battn_ref,
               idx_ref, w_ref):
    b = pl.program_id(0)
    q = q_ref[0]

    offs = jnp.dot(q, woff_ref[...], preferred_element_type=jnp.float32)
    offs = offs + boff_ref[...]
    off_x = offs[:, :128]
    off_y = offs[:, 128:]

    logits = jnp.dot(q, wattn_ref[...], preferred_element_type=jnp.float32)
    logits = logits + battn_ref[...]
    e = jnp.exp(logits)
    ri = lax.broadcasted_iota(jnp.int32, (128, 128), 0) // SP
    ci = lax.broadcasted_iota(jnp.int32, (128, 128), 1) // SP
    bd = (ri == ci).astype(jnp.float32)
    denom = jnp.dot(e, bd, preferred_element_type=jnp.float32)
    attn = e / denom

    lane = lax.broadcasted_iota(jnp.int32, (QB, 128), 1)
    h_l = lane // SP
    lvl = (lane // 4) % 4
    w_i = jnp.where(lvl == 0, SHAPES[0][1],
                    jnp.where(lvl == 1, SHAPES[1][1],
                              jnp.where(lvl == 2, SHAPES[2][1], SHAPES[3][1])))
    h_i = w_i
    start_v = jnp.where(lvl == 0, STARTS[0],
                        jnp.where(lvl == 1, STARTS[1],
                                  jnp.where(lvl == 2, STARTS[2], STARTS[3])))
    w_f = w_i.astype(jnp.float32)
    h_f = h_i.astype(jnp.float32)

    r = r_ref[0]
    rx = r[:, 0:1]
    ry = r[:, 1:2]
    rw = r[:, 2:3]
    rh = r[:, 3:4]

    loc_x = rx + off_x * 0.125 * rw
    loc_y = ry + off_y * 0.125 * rh
    x = loc_x * w_f - 0.5
    y = loc_y * h_f - 0.5
    x0f = jnp.floor(x)
    y0f = jnp.floor(y)
    fx = x - x0f
    fy = y - y0f
    x0 = x0f.astype(jnp.int32)
    y0 = y0f.astype(jnp.int32)

    base_row = (b * LEN_V + start_v) * H + h_l
    idx_parts = []
    w_parts = []
    for dx, dy in ((0, 0), (1, 0), (0, 1), (1, 1)):
        xi = x0 + dx
        yi = y0 + dy
        valid = (xi >= 0) & (xi < w_i) & (yi >= 0) & (yi < h_i)
        xc = jnp.clip(xi, 0, w_i - 1)
        yc = jnp.clip(yi, 0, h_i - 1)
        flat = base_row + (yc * w_i + xc) * H
        wx = fx if dx else (1.0 - fx)
        wy = fy if dy else (1.0 - fy)
        wgt = attn * wx * wy * valid.astype(jnp.float32)
        idx_parts.append(flat)
        w_parts.append(wgt)
    idx_ref[0] = jnp.concatenate(idx_parts, axis=-1)
    w_ref[0] = jnp.concatenate(w_parts, axis=-1)


def _prep(query, rp, woff_p, boff_p, wattn, battn2):
    bs, lq, _ = query.shape
    grid = (bs, lq // QB)
    return pl.pallas_call(
        _prep_body,
        grid=grid,
        in_specs=[
            pl.BlockSpec((1, QB, EMBED), lambda b, i: (b, i, 0)),
            pl.BlockSpec((1, QB, 4), lambda b, i: (b, i, 0)),
            pl.BlockSpec((EMBED, EMBED), lambda b, i: (0, 0)),
            pl.BlockSpec((1, EMBED), lambda b, i: (0, 0)),
            pl.BlockSpec((EMBED, 128), lambda b, i: (0, 0)),
            pl.BlockSpec((1, 128), lambda b, i: (0, 0)),
        ],
        out_specs=[
            pl.BlockSpec((1, QB, 512), lambda b, i: (b, i, 0)),
            pl.BlockSpec((1, QB, 512), lambda b, i: (b, i, 0)),
        ],
        out_shape=[
            jax.ShapeDtypeStruct((bs, lq, 512), jnp.int32),
            jax.ShapeDtypeStruct((bs, lq, 512), jnp.float32),
        ],
    )(query, rp, woff_p, boff_p, wattn, battn2)


NSTEPS = PAIRS_PER_W // NQ


def _bag_body(value_ref, idx_hbm, w_hbm, out_hbm, idx_v, w_v, rows_v, out_v,
              ss0, ss1, ss2, ss3, gs0, gs1, os0, os1):
    ssem = (ss0, ss1, ss2, ss3)
    gsem = (gs0, gs1)
    osem = (os0, os1)
    nc = lax.axis_index("c")
    ns = lax.axis_index("s")
    wid = ns * 2 + nc
    pair0 = wid * PAIRS_PER_W

    def stage(s, sb, sem):
        g = pair0 + NQ * s
        return (
            pltpu.make_async_copy(idx_hbm.at[pl.ds(4 * g, 8)],
                                  idx_v.at[pl.ds(sb * 8, 8)], sem),
            pltpu.make_async_copy(w_hbm.at[pl.ds(g * 512, 1024)],
                                  w_v.at[pl.ds(sb * 1024, 1024)], sem),
        )

    def gathers(sb, rb, sem):
        return [pltpu.make_async_copy(value_ref.at[idx_v.at[sb * 8 + j]],
                                      rows_v.at[rb * 8 + j], sem)
                for j in range(8)]

    def out_dma(s, rb, sem):
        g = pair0 + NQ * s
        return pltpu.make_async_copy(out_v.at[pl.ds(rb * 512, 512)],
                                     out_hbm.at[pl.ds(g * 256, 512)], sem)

    def compute(s, sb, rb):
        def bag(i, _):
            q_l = i // 8
            h = i % 8
            wsplat = jnp.full((16,), sb * 1024 + q_l * 512 + h * 16,
                              jnp.int32)
            acc0 = jnp.zeros((16,), jnp.float32)
            acc1 = jnp.zeros((16,), jnp.float32)
            for k in range(64):
                c, lp = k // 16, k % 16
                wb = plsc.load_gather(w_v, [wsplat + (c * 128 + lp)])
                j = rb * 8 + q_l * 4 + c
                r = h * 16 + lp
                d0 = rows_v[j, r, pl.ds(0, 16)]
                d1 = rows_v[j, r, pl.ds(16, 16)]
                acc0 = acc0 + wb * d0
                acc1 = acc1 + wb * d1
            ob = rb * 512 + i * 32
            out_v[pl.ds(ob, 16)] = acc0
            out_v[pl.ds(ob + 16, 16)] = acc1
            return 0

        lax.fori_loop(0, 16, bag, 0)

    for d in stage(0, 0, ssem[0]):
        d.start()
        d.wait()
    for d in gathers(0, 0, gsem[0]):
        d.start()
    for d in stage(1, 1, ssem[1]):
        d.start()

    def body4(tt, _):
        for i in range(4):
            s = tt * 4 + i
            sb1 = (i + 1) % 4
            sb2 = (i + 2) % 4
            rb = i % 2
            rb1 = (i + 1) % 2

            @pl.when(s + 1 < NSTEPS)
            def _():
                for d in stage(s + 1, sb1, ssem[sb1]):
                    d.wait()
                for d in gathers(sb1, rb1, gsem[rb1]):
                    d.start()

            for d in gathers(i, rb, gsem[rb]):
                d.wait()

            @pl.when(s + 2 < NSTEPS)
            def _():
                for d in stage(s + 2, sb2, ssem[sb2]):
                    d.start()

            @pl.when(s >= 2)
            def _():
                out_dma(s, rb, osem[rb]).wait()

            compute(s, i, rb)
            out_dma(s, rb, osem[rb]).start()
        return 0

    lax.fori_loop(0, NSTEPS // 4, body4, 0)
    out_dma(0, 0, osem[0]).wait()
    out_dma(0, 1, osem[1]).wait()


def _bag_sum(value2, idx2, w2):
    mesh = plsc.VectorSubcoreMesh(core_axis_name="c", subcore_axis_name="s")
    f = pl.kernel(
        _bag_body,
        out_type=jax.ShapeDtypeStruct((16384 * H * HD,), jnp.float32),
        mesh=mesh,
        scratch_types=[
            pltpu.VMEM((32, 128), jnp.int32),
            pltpu.VMEM((4096,), jnp.float32),
            pltpu.VMEM((16, 128, HD), jnp.float32),
            pltpu.VMEM((1024,), jnp.float32),
            pltpu.SemaphoreType.DMA,
            pltpu.SemaphoreType.DMA,
            pltpu.SemaphoreType.DMA,
            pltpu.SemaphoreType.DMA,
            pltpu.SemaphoreType.DMA,
            pltpu.SemaphoreType.DMA,
            pltpu.SemaphoreType.DMA,
            pltpu.SemaphoreType.DMA,
        ],
        compiler_params=pltpu.CompilerParams(needs_layout_passes=False,
                                             use_tc_tiling_on_sc=False),
    )
    return f(value2, idx2, w2)


def kernel(query, reference_points, value, value_spatial_shapes,
           W_off, b_off, W_attn, b_attn):
    bs, lq = query.shape[:2]
    woff_p = jnp.concatenate([W_off[:, 0::2], W_off[:, 1::2]], axis=1)
    boff_p = jnp.concatenate([b_off[0::2], b_off[1::2]])[None, :]
    battn2 = b_attn[None, :]
    rp = reference_points.reshape(bs, lq, 4)

    idx, w = _prep(query, rp, woff_p, boff_p, W_attn, battn2)

    value2 = value.reshape(bs * LEN_V * H, HD)
    idx2 = idx.reshape(bs * 4 * lq, 128)
    w2 = w.reshape(bs * 4 * lq * 128)
    out2 = _bag_sum(value2, idx2, w2)
    return out2.reshape(bs, lq, EMBED)
